# baseline (speedup 1.0000x reference)
import jax
import jax.numpy as jnp
from jax.experimental import pallas as pl
from jax.experimental.pallas import tpu as pltpu

OUT_H, OUT_W = 448, 448
B_TOTAL = 128
H, W = 512, 512
BLK_B = 8
IN_R = OUT_H + 8


def _offsets(h_range, w_range):
    kk = jax.random.key(42)
    kx, ky = jax.random.split(kk)
    xo = jax.random.randint(kx, (B_TOTAL,), 0, h_range, dtype=jnp.int32)
    yo = jax.random.randint(ky, (B_TOTAL,), 0, w_range, dtype=jnp.int32)
    return xo, yo


def _crop_body(xo_ref, yo_ref, x_hbm, o_ref, ib0, ib1, sem):
    g = pl.program_id(0)
    n = pl.num_programs(0)

    def in_copy(gg, buf, sl, i):
        b = gg * BLK_B + i
        a8 = pl.multiple_of((xo_ref[b] // 8) * 8, 8)
        return pltpu.make_async_copy(
            x_hbm.at[b, 0, pl.ds(a8, IN_R), :], buf.at[i], sem.at[sl, i]
        )

    def start_block(gg, buf, sl):
        for i in range(BLK_B):
            in_copy(gg, buf, sl, i).start()

    def process(buf, sl):
        for i in range(BLK_B):
            b = g * BLK_B + i
            in_copy(g, buf, sl, i).wait()
            q = xo_ref[b] - (xo_ref[b] // 8) * 8
            img = buf[i]
            img = pltpu.roll(img, -yo_ref[b], 1)
            img = pltpu.roll(img, -q, 0)
            o_ref[i, 0] = img[:OUT_H, :OUT_W]

    @pl.when(g == 0)
    def _():
        start_block(0, ib0, 0)

    @pl.when(jnp.logical_and(g + 1 < n, g % 2 == 0))
    def _():
        start_block(g + 1, ib1, 1)

    @pl.when(jnp.logical_and(g + 1 < n, g % 2 == 1))
    def _():
        start_block(g + 1, ib0, 0)

    @pl.when(g % 2 == 0)
    def _():
        process(ib0, 0)

    @pl.when(g % 2 == 1)
    def _():
        process(ib1, 1)


def kernel(x):
    B, C, _, _ = x.shape
    grid_spec = pltpu.PrefetchScalarGridSpec(
        num_scalar_prefetch=2,
        grid=(B // BLK_B,),
        in_specs=[pl.BlockSpec(memory_space=pltpu.MemorySpace.HBM)],
        out_specs=pl.BlockSpec(
            (BLK_B, 1, OUT_H, OUT_W), lambda b, xo, yo: (b, 0, 0, 0)
        ),
        scratch_shapes=[
            pltpu.VMEM((BLK_B, IN_R, W), x.dtype),
            pltpu.VMEM((BLK_B, IN_R, W), x.dtype),
            pltpu.SemaphoreType.DMA((2, BLK_B)),
        ],
    )
    xo, yo = _offsets(H - OUT_H, W - OUT_W)
    out = pl.pallas_call(
        _crop_body,
        grid_spec=grid_spec,
        out_shape=jax.ShapeDtypeStruct((B, C, OUT_H, OUT_W), x.dtype),
    )(xo, yo, x)
    return out


# speedup vs baseline: 1.0739x; 1.0739x over previous
import jax
import jax.numpy as jnp
from jax.experimental import pallas as pl
from jax.experimental.pallas import tpu as pltpu

OUT_H, OUT_W = 448, 448
B_TOTAL = 128
H, W = 512, 512
BLK_B = 8
IN_R = OUT_H + 8


def _offsets(h_range, w_range):
    kk = jax.random.key(42)
    kx, ky = jax.random.split(kk)
    xo = jax.random.randint(kx, (B_TOTAL,), 0, h_range, dtype=jnp.int32)
    yo = jax.random.randint(ky, (B_TOTAL,), 0, w_range, dtype=jnp.int32)
    return xo, yo


def _crop_body(xo_ref, yo_ref, x_hbm, o_ref, ib0, ib1, sem):
    g = pl.program_id(0)
    n = pl.num_programs(0)

    def in_copy(gg, buf, sl, i):
        b = gg * BLK_B + i
        a8 = pl.multiple_of((xo_ref[b] // 8) * 8, 8)
        return pltpu.make_async_copy(
            x_hbm.at[b, 0, pl.ds(a8, IN_R), :],
            buf.at[i, pl.ds(0, IN_R), :],
            sem.at[sl, i],
        )

    def start_block(gg, buf, sl):
        for i in range(BLK_B):
            in_copy(gg, buf, sl, i).start()

    def process(buf, sl):
        for i in range(BLK_B):
            b = g * BLK_B + i
            in_copy(g, buf, sl, i).wait()
            q = xo_ref[b] - (xo_ref[b] // 8) * 8
            img = buf[i]
            img = pltpu.roll(img, -q, 0)
            img = pltpu.roll(img, -yo_ref[b], 1)
            o_ref[i, 0] = img[:OUT_H, :OUT_W]

    @pl.when(g == 0)
    def _():
        start_block(0, ib0, 0)

    @pl.when(jnp.logical_and(g + 1 < n, g % 2 == 0))
    def _():
        start_block(g + 1, ib1, 1)

    @pl.when(jnp.logical_and(g + 1 < n, g % 2 == 1))
    def _():
        start_block(g + 1, ib0, 0)

    @pl.when(g % 2 == 0)
    def _():
        process(ib0, 0)

    @pl.when(g % 2 == 1)
    def _():
        process(ib1, 1)


def kernel(x):
    B, C, _, _ = x.shape
    grid_spec = pltpu.PrefetchScalarGridSpec(
        num_scalar_prefetch=2,
        grid=(B // BLK_B,),
        in_specs=[pl.BlockSpec(memory_space=pltpu.MemorySpace.HBM)],
        out_specs=pl.BlockSpec(
            (BLK_B, 1, OUT_H, OUT_W), lambda b, xo, yo: (b, 0, 0, 0)
        ),
        scratch_shapes=[
            pltpu.VMEM((BLK_B, H, W), x.dtype),
            pltpu.VMEM((BLK_B, H, W), x.dtype),
            pltpu.SemaphoreType.DMA((2, BLK_B)),
        ],
    )
    xo, yo = _offsets(H - OUT_H, W - OUT_W)
    out = pl.pallas_call(
        _crop_body,
        grid_spec=grid_spec,
        out_shape=jax.ShapeDtypeStruct((B, C, OUT_H, OUT_W), x.dtype),
    )(xo, yo, x)
    return out
